# baseline (speedup 1.0000x reference)
import functools

import jax
import jax.numpy as jnp
from jax import lax
from jax.experimental import pallas as pl
from jax.experimental.pallas import tpu as pltpu
from jax.experimental.pallas import tpu_sc as plsc

GLOVE_X_MAX = 100.0
GLOVE_ALPHA = 0.75

_LN2 = 0.6931471805599453
_SQRT2 = 1.4142135623730951
_LN_XMAX = 4.605170185988091

_NC = 2
_NS = 16
_NW = _NC * _NS
_L = 16
_GCHUNK = 128


def _ln(x):
    bits = plsc.bitcast(x, jnp.int32)
    e = (bits >> 23) - 127
    m = plsc.bitcast((bits & 0x007FFFFF) | 0x3F800000, jnp.float32)
    big = m > _SQRT2
    m = jnp.where(big, m * 0.5, m)
    e = e + big.astype(jnp.int32)
    s = (m - 1.0) / (m + 1.0)
    s2 = s * s
    lnm = s * (2.0 + s2 * (0.6666666666 + s2 * (0.4 + s2 * 0.2857142857)))
    return lnm + e.astype(jnp.float32) * _LN2


def _make_sc_call(B, D):
    C = B // _NW
    G = C // _L
    NCH = C // _GCHUNK
    mesh = plsc.VectorSubcoreMesh(core_axis_name="c", subcore_axis_name="s")

    @functools.partial(
        pl.kernel,
        mesh=mesh,
        compiler_params=pltpu.CompilerParams(
            needs_layout_passes=False, use_tc_tiling_on_sc=False),
        out_type=jax.ShapeDtypeStruct((_NW, _L), jnp.float32),
        scratch_types=[
            pltpu.VMEM((NCH, _GCHUNK), jnp.int32),
            pltpu.VMEM((NCH, _GCHUNK), jnp.int32),
            pltpu.VMEM((C,), jnp.float32),
            pltpu.VMEM((D, C), jnp.float32),
            pltpu.VMEM((D, C), jnp.float32),
            pltpu.VMEM((_L,), jnp.float32),
            pltpu.SemaphoreType.DMA,
        ],
    )
    def sc_call(i_hbm, j_hbm, x_hbm, wt_hbm, wtt_hbm, out_hbm,
                ii_v, jj_v, x_v, wi_v, wj_v, acc_v, sem):
        wid = lax.axis_index("s") * _NC + lax.axis_index("c")
        base = wid * C
        for k in range(NCH):
            pltpu.sync_copy(i_hbm.at[pl.ds(base + k * _GCHUNK, _GCHUNK)],
                            ii_v.at[k])
            pltpu.sync_copy(j_hbm.at[pl.ds(base + k * _GCHUNK, _GCHUNK)],
                            jj_v.at[k])
        pltpu.sync_copy(x_hbm.at[pl.ds(base, C)], x_v)

        def fetch_d(d):
            copies = []
            for k in range(NCH):
                dst = pl.ds(k * _GCHUNK, _GCHUNK)
                copies.append(pltpu.async_copy(
                    wt_hbm.at[d].at[ii_v.at[k]], wi_v.at[d, dst], sem))
                copies.append(pltpu.async_copy(
                    wtt_hbm.at[d].at[jj_v.at[k]], wj_v.at[d, dst], sem))
            for cp in copies:
                cp.wait()

        pl.loop(0, D)(fetch_d)

        def group(g, acc):
            gbase = g * _L
            sl = pl.ds(gbase, _L)
            dots = wi_v[0, sl] * wj_v[0, sl]
            for d in range(1, D):
                dots = dots + wi_v[d, sl] * wj_v[d, sl]
            xg = x_v[sl]
            lnx = _ln(xg)
            lnw = jnp.minimum(lnx - _LN_XMAX, 0.0)
            weight = jnp.exp(jnp.float32(GLOVE_ALPHA) * lnw)
            diff = dots - lnx
            return acc + weight * diff * diff

        acc = lax.fori_loop(0, G, group, jnp.zeros((_L,), jnp.float32))
        acc_v[...] = acc
        pltpu.sync_copy(acc_v, out_hbm.at[wid])

    return sc_call


def kernel(i_idx, j_idx, x_ij, W, W_tilde, b, b_tilde):
    B = x_ij.shape[0]
    D = W.shape[1]
    W, W_tilde = lax.optimization_barrier((W, W_tilde))
    sc_call = _make_sc_call(B, D)
    partials = sc_call(i_idx.astype(jnp.int32), j_idx.astype(jnp.int32),
                       x_ij, W.T, W_tilde.T)
    return jnp.sum(partials) / jnp.float32(B)


# speedup vs baseline: 20.9867x; 20.9867x over previous
import functools

import jax
import jax.numpy as jnp
from jax import lax
from jax.experimental import pallas as pl
from jax.experimental.pallas import tpu as pltpu
from jax.experimental.pallas import tpu_sc as plsc

GLOVE_X_MAX = 100.0
GLOVE_ALPHA = 0.75

_LN2 = 0.6931471805599453
_SQRT2 = 1.4142135623730951
_LN_XMAX = 4.605170185988091

_NC = 2
_NS = 16
_NW = _NC * _NS
_L = 16
_WAVE = 4
_RB = 8


def _ln(x):
    bits = plsc.bitcast(x, jnp.int32)
    e = (bits >> 23) - 127
    m = plsc.bitcast((bits & 0x007FFFFF) | 0x3F800000, jnp.float32)
    big = m > _SQRT2
    m = jnp.where(big, m * 0.5, m)
    e = e + big.astype(jnp.int32)
    s = (m - 1.0) / (m + 1.0)
    s2 = s * s
    lnm = s * (2.0 + s2 * (0.6666666666 + s2 * (0.4 + s2 * 0.2857142857)))
    return lnm + e.astype(jnp.float32) * _LN2


def _make_sc_call(B, D):
    C = B // _NW
    G = C // _L
    WPG = _L // _WAVE
    mesh = plsc.VectorSubcoreMesh(core_axis_name="c", subcore_axis_name="s")

    @functools.partial(
        pl.kernel,
        mesh=mesh,
        compiler_params=pltpu.CompilerParams(needs_layout_passes=False),
        out_type=jax.ShapeDtypeStruct((_NW, _L), jnp.float32),
        scratch_types=[
            pltpu.VMEM((C + _L,), jnp.int32),
            pltpu.VMEM((C + _L,), jnp.int32),
            pltpu.VMEM((C,), jnp.float32),
            pltpu.VMEM((_WAVE * _RB, 8, 128), jnp.float32),
            pltpu.VMEM((_WAVE * _RB, 8, 128), jnp.float32),
            pltpu.VMEM((_L,), jnp.float32),
            pltpu.SemaphoreType.DMA,
        ],
    )
    def sc_call(i_hbm, j_hbm, x_hbm, w_hbm, wt_hbm, out_hbm,
                ii_v, jj_v, x_v, bufi, bufj, acc_v, sem):
        wid = lax.axis_index("s") * _NC + lax.axis_index("c")
        base = wid * C
        pltpu.sync_copy(i_hbm.at[pl.ds(base, C)], ii_v.at[pl.ds(0, C)])
        pltpu.sync_copy(j_hbm.at[pl.ds(base, C)], jj_v.at[pl.ds(0, C)])
        pltpu.sync_copy(x_hbm.at[pl.ds(base, C)], x_v)

        iota = lax.iota(jnp.int32, _L)
        slot_c = [((16 * k + iota) >> 3) for k in range(D // _L)]
        row_c = [((16 * k + iota) & 7) for k in range(D // _L)]

        def pair_dot(q, vi, vj):
            ci = vi & 127
            cj = vj & 127
            t = None
            for k in range(D // _L):
                gi = plsc.load_gather(
                    bufi, [slot_c[k] + q * _RB, row_c[k],
                           jnp.full((_L,), 1, jnp.int32) * ci])
                gj = plsc.load_gather(
                    bufj, [slot_c[k] + q * _RB, row_c[k],
                           jnp.full((_L,), 1, jnp.int32) * cj])
                t = gi * gj if t is None else t + gi * gj
            for sh in (1, 2, 4, 8):
                t = t + t.at[iota ^ sh].get(mode="promise_in_bounds")
            return t

        def group(g, acc):
            gbase = g * _L
            dots = jnp.zeros((_L,), jnp.float32)
            for w in range(WPG):
                iv = ii_v[pl.ds(gbase + w * _WAVE, _L)]
                jv = jj_v[pl.ds(gbase + w * _WAVE, _L)]
                copies = []
                for q in range(_WAVE):
                    vi = iv[q]
                    vj = jv[q]
                    vbi = pl.multiple_of((vi >> 7) * 128, 128)
                    vbj = pl.multiple_of((vj >> 7) * 128, 128)
                    for r in range(_RB):
                        copies.append(pltpu.async_copy(
                            w_hbm.at[r, :, pl.ds(vbi, 128)],
                            bufi.at[q * _RB + r], sem))
                        copies.append(pltpu.async_copy(
                            wt_hbm.at[r, :, pl.ds(vbj, 128)],
                            bufj.at[q * _RB + r], sem))
                for cp in copies:
                    cp.wait()
                for q in range(_WAVE):
                    t = pair_dot(q, iv[q], jv[q])
                    dots = jnp.where(iota == (w * _WAVE + q), t, dots)
            xg = x_v[pl.ds(gbase, _L)]
            lnx = _ln(xg)
            lnw = jnp.minimum(lnx - _LN_XMAX, 0.0)
            weight = jnp.exp(jnp.float32(GLOVE_ALPHA) * lnw)
            diff = dots - lnx
            return acc + weight * diff * diff

        acc = lax.fori_loop(0, G, group, jnp.zeros((_L,), jnp.float32))
        acc_v[...] = acc
        pltpu.sync_copy(acc_v, out_hbm.at[wid])

    return sc_call


def kernel(i_idx, j_idx, x_ij, W, W_tilde, b, b_tilde):
    B = x_ij.shape[0]
    D = W.shape[1]
    sc_call = _make_sc_call(B, D)
    partials = sc_call(i_idx.astype(jnp.int32), j_idx.astype(jnp.int32),
                       x_ij, W.T.reshape(8, 8, W.shape[0]),
                       W_tilde.T.reshape(8, 8, W.shape[0]))
    return jnp.sum(partials) / jnp.float32(B)


# speedup vs baseline: 21.5080x; 1.0248x over previous
import functools

import jax
import jax.numpy as jnp
from jax import lax
from jax.experimental import pallas as pl
from jax.experimental.pallas import tpu as pltpu
from jax.experimental.pallas import tpu_sc as plsc

GLOVE_X_MAX = 100.0
GLOVE_ALPHA = 0.75

_LN2 = 0.6931471805599453
_SQRT2 = 1.4142135623730951
_LN_XMAX = 4.605170185988091

_NC = 2
_NS = 16
_NW = _NC * _NS
_L = 16
_RB = 8


def _ln(x):
    bits = plsc.bitcast(x, jnp.int32)
    e = (bits >> 23) - 127
    m = plsc.bitcast((bits & 0x007FFFFF) | 0x3F800000, jnp.float32)
    big = m > _SQRT2
    m = jnp.where(big, m * 0.5, m)
    e = e + big.astype(jnp.int32)
    s = (m - 1.0) / (m + 1.0)
    s2 = s * s
    lnm = s * (2.0 + s2 * (0.6666666666 + s2 * (0.4 + s2 * 0.2857142857)))
    return lnm + e.astype(jnp.float32) * _LN2


def _make_sc_call(B, D):
    C = B // _NW
    G = C // _L
    mesh = plsc.VectorSubcoreMesh(core_axis_name="c", subcore_axis_name="s")

    @functools.partial(
        pl.kernel,
        mesh=mesh,
        compiler_params=pltpu.CompilerParams(needs_layout_passes=False),
        out_type=jax.ShapeDtypeStruct((_NW, _L), jnp.float32),
        scratch_types=[
            pltpu.VMEM((C,), jnp.int32),
            pltpu.VMEM((C,), jnp.int32),
            pltpu.VMEM((C,), jnp.float32),
            pltpu.VMEM((2 * _L, 8, 128), jnp.float32),
            pltpu.VMEM((2 * _L, 8, 128), jnp.float32),
            pltpu.VMEM((_L,), jnp.float32),
            pltpu.SemaphoreType.DMA,
            pltpu.SemaphoreType.DMA,
        ],
    )
    def sc_call(i_hbm, j_hbm, x_hbm, w_hbm, wt_hbm, out_hbm,
                ii_v, jj_v, x_v, bufi, bufj, acc_v, sem0, sem1):
        wid = lax.axis_index("s") * _NC + lax.axis_index("c")
        base = wid * C
        pltpu.sync_copy(i_hbm.at[pl.ds(base, C)], ii_v)
        pltpu.sync_copy(j_hbm.at[pl.ds(base, C)], jj_v)
        pltpu.sync_copy(x_hbm.at[pl.ds(base, C)], x_v)

        iota = lax.iota(jnp.int32, _L)
        sems = (sem0, sem1)

        def group(g, acc):
            gbase = g * _L
            iv = ii_v[pl.ds(gbase, _L)]
            jv = jj_v[pl.ds(gbase, _L)]
            civ = iv & 127
            cjv = jv & 127
            vbi = (iv >> 7) * 128
            vbj = (jv >> 7) * 128

            def fire(r):
                pb = (r & 1) * _L
                sem = sems[r & 1]
                copies = []
                for q in range(_L):
                    oi = pl.multiple_of(vbi[q], 128)
                    oj = pl.multiple_of(vbj[q], 128)
                    copies.append(pltpu.async_copy(
                        w_hbm.at[r, :, pl.ds(oi, 128)],
                        bufi.at[pb + q], sem))
                    copies.append(pltpu.async_copy(
                        wt_hbm.at[r, :, pl.ds(oj, 128)],
                        bufj.at[pb + q], sem))
                return copies

            def extract(r, dots):
                pb = (r & 1) * _L
                slot = iota + pb
                for dr in range(8):
                    drv = jnp.full((_L,), dr, jnp.int32)
                    gi = plsc.load_gather(bufi, [slot, drv, civ])
                    gj = plsc.load_gather(bufj, [slot, drv, cjv])
                    dots = dots + gi * gj
                return dots

            dots = jnp.zeros((_L,), jnp.float32)
            inflight = fire(0)
            for r in range(_RB):
                nxt = fire(r + 1) if r + 1 < _RB else []
                for cp in inflight:
                    cp.wait()
                dots = extract(r, dots)
                inflight = nxt

            xg = x_v[pl.ds(gbase, _L)]
            lnx = _ln(xg)
            lnw = jnp.minimum(lnx - _LN_XMAX, 0.0)
            weight = jnp.exp(jnp.float32(GLOVE_ALPHA) * lnw)
            diff = dots - lnx
            return acc + weight * diff * diff

        acc = lax.fori_loop(0, G, group, jnp.zeros((_L,), jnp.float32))
        acc_v[...] = acc
        pltpu.sync_copy(acc_v, out_hbm.at[wid])

    return sc_call


def kernel(i_idx, j_idx, x_ij, W, W_tilde, b, b_tilde):
    B = x_ij.shape[0]
    D = W.shape[1]
    sc_call = _make_sc_call(B, D)
    partials = sc_call(i_idx.astype(jnp.int32), j_idx.astype(jnp.int32),
                       x_ij, W.T.reshape(8, 8, W.shape[0]),
                       W_tilde.T.reshape(8, 8, W.shape[0]))
    return jnp.sum(partials) / jnp.float32(B)


# speedup vs baseline: 24.1962x; 1.1250x over previous
import functools

import jax
import jax.numpy as jnp
from jax import lax
from jax.experimental import pallas as pl
from jax.experimental.pallas import tpu as pltpu
from jax.experimental.pallas import tpu_sc as plsc

GLOVE_X_MAX = 100.0
GLOVE_ALPHA = 0.75

_LN2 = 0.6931471805599453
_SQRT2 = 1.4142135623730951
_LN_XMAX = 4.605170185988091

_NC = 2
_NS = 16
_NW = _NC * _NS
_L = 16
_RB = 8
_RING = 3


def _ln(x):
    bits = plsc.bitcast(x, jnp.int32)
    e = (bits >> 23) - 127
    m = plsc.bitcast((bits & 0x007FFFFF) | 0x3F800000, jnp.float32)
    big = m > _SQRT2
    m = jnp.where(big, m * 0.5, m)
    e = e + big.astype(jnp.int32)
    s = (m - 1.0) / (m + 1.0)
    s2 = s * s
    lnm = s * (2.0 + s2 * (0.6666666666 + s2 * (0.4 + s2 * 0.2857142857)))
    return lnm + e.astype(jnp.float32) * _LN2


def _make_sc_call(B, D):
    C = B // _NW
    G = C // _L
    T = G * _RB
    mesh = plsc.VectorSubcoreMesh(core_axis_name="c", subcore_axis_name="s")

    @functools.partial(
        pl.kernel,
        mesh=mesh,
        compiler_params=pltpu.CompilerParams(needs_layout_passes=False),
        out_type=jax.ShapeDtypeStruct((_NW, _L), jnp.float32),
        scratch_types=[
            pltpu.VMEM((C,), jnp.int32),
            pltpu.VMEM((C,), jnp.int32),
            pltpu.VMEM((C,), jnp.float32),
            pltpu.VMEM((_RING * _L, 8, 128), jnp.float32),
            pltpu.VMEM((_RING * _L, 8, 128), jnp.float32),
            pltpu.VMEM((_L,), jnp.float32),
            pltpu.SemaphoreType.DMA,
            pltpu.SemaphoreType.DMA,
            pltpu.SemaphoreType.DMA,
        ],
    )
    def sc_call(i_hbm, j_hbm, x_hbm, w_hbm, wt_hbm, out_hbm,
                ii_v, jj_v, x_v, bufi, bufj, acc_v, sem0, sem1, sem2):
        wid = lax.axis_index("s") * _NC + lax.axis_index("c")
        base = wid * C
        pltpu.sync_copy(i_hbm.at[pl.ds(base, C)], ii_v)
        pltpu.sync_copy(j_hbm.at[pl.ds(base, C)], jj_v)
        pltpu.sync_copy(x_hbm.at[pl.ds(base, C)], x_v)

        iota = lax.iota(jnp.int32, _L)
        sems = (sem0, sem1, sem2)

        def fire(t, u):
            g = t // _RB
            r = t % _RB
            slot = u * _L
            sem = sems[u]
            iv = ii_v[pl.ds(g * _L, _L)]
            jv = jj_v[pl.ds(g * _L, _L)]
            vbi = (iv >> 7) * 128
            vbj = (jv >> 7) * 128
            for q in range(_L):
                oi = pl.multiple_of(vbi[q], 128)
                oj = pl.multiple_of(vbj[q], 128)
                pltpu.async_copy(w_hbm.at[r, :, pl.ds(oi, 128)],
                                 bufi.at[slot + q], sem)
                pltpu.async_copy(wt_hbm.at[r, :, pl.ds(oj, 128)],
                                 bufj.at[slot + q], sem)

        def drain(u):
            slot = u * _L
            sem = sems[u]
            for q in range(_L):
                pltpu.make_async_copy(w_hbm.at[0, :, pl.ds(0, 128)],
                                      bufi.at[slot + q], sem).wait()
                pltpu.make_async_copy(w_hbm.at[0, :, pl.ds(0, 128)],
                                      bufj.at[slot + q], sem).wait()

        def do_phase(t, u, dots, acc):
            drain(u)
            g = t // _RB
            slot = u * _L + iota
            iv = ii_v[pl.ds(g * _L, _L)]
            jv = jj_v[pl.ds(g * _L, _L)]
            civ = iv & 127
            cjv = jv & 127
            for dr in range(8):
                drv = jnp.full((_L,), dr, jnp.int32)
                gi = plsc.load_gather(bufi, [slot, drv, civ])
                gj = plsc.load_gather(bufj, [slot, drv, cjv])
                dots = dots + gi * gj

            xg = x_v[pl.ds(g * _L, _L)]
            lnx = _ln(xg)
            lnw = jnp.minimum(lnx - _LN_XMAX, 0.0)
            weight = jnp.exp(jnp.float32(GLOVE_ALPHA) * lnw)
            diff = dots - lnx
            contrib = weight * diff * diff
            last = jnp.full((_L,), (t % _RB) == (_RB - 1))
            acc = acc + jnp.where(last, contrib, 0.0)
            dots = jnp.where(last, 0.0, dots)
            return dots, acc

        fire(0, 0)
        fire(1, 1)

        def body(i, carry):
            dots, acc = carry
            t0 = i * _RING
            for u in range(_RING):
                t = t0 + u

                @pl.when(t + 2 < T)
                def _(t=t, u=u):
                    fire(t + 2, (u + 2) % _RING)

                dots, acc = do_phase(t, u, dots, acc)
            return dots, acc

        nfull = (T - 1) // _RING
        dots, acc = lax.fori_loop(
            0, nfull, body,
            (jnp.zeros((_L,), jnp.float32), jnp.zeros((_L,), jnp.float32)))
        for t in range(nfull * _RING, T):
            _, acc = do_phase(t, t % _RING, dots, acc)
        acc_v[...] = acc
        pltpu.sync_copy(acc_v, out_hbm.at[wid])

    return sc_call


def kernel(i_idx, j_idx, x_ij, W, W_tilde, b, b_tilde):
    B = x_ij.shape[0]
    D = W.shape[1]
    sc_call = _make_sc_call(B, D)
    partials = sc_call(i_idx.astype(jnp.int32), j_idx.astype(jnp.int32),
                       x_ij, W.T.reshape(8, 8, W.shape[0]),
                       W_tilde.T.reshape(8, 8, W.shape[0]))
    return jnp.sum(partials) / jnp.float32(B)


# speedup vs baseline: 24.3024x; 1.0044x over previous
import functools

import jax
import jax.numpy as jnp
from jax import lax
from jax.experimental import pallas as pl
from jax.experimental.pallas import tpu as pltpu
from jax.experimental.pallas import tpu_sc as plsc

GLOVE_X_MAX = 100.0
GLOVE_ALPHA = 0.75

_LN2 = 0.6931471805599453
_SQRT2 = 1.4142135623730951
_LN_XMAX = 4.605170185988091

_NC = 2
_NS = 16
_NW = _NC * _NS
_L = 16
_RB = 8
_RING = 3


def _ln(x):
    bits = plsc.bitcast(x, jnp.int32)
    e = (bits >> 23) - 127
    m = plsc.bitcast((bits & 0x007FFFFF) | 0x3F800000, jnp.float32)
    big = m > _SQRT2
    m = jnp.where(big, m * 0.5, m)
    e = e + big.astype(jnp.int32)
    s = (m - 1.0) / (m + 1.0)
    s2 = s * s
    lnm = s * (2.0 + s2 * (0.6666666666 + s2 * (0.4 + s2 * 0.2857142857)))
    return lnm + e.astype(jnp.float32) * _LN2


def _make_sc_call(B, D):
    C = B // _NW
    G = C // _L
    T = G * _RB
    mesh = plsc.VectorSubcoreMesh(core_axis_name="c", subcore_axis_name="s")

    @functools.partial(
        pl.kernel,
        mesh=mesh,
        compiler_params=pltpu.CompilerParams(needs_layout_passes=False),
        out_type=jax.ShapeDtypeStruct((_NW, _L), jnp.float32),
        scratch_types=[
            pltpu.VMEM((C,), jnp.int32),
            pltpu.VMEM((C,), jnp.int32),
            pltpu.VMEM((C,), jnp.float32),
            pltpu.VMEM((_RING * _L, 8, 128), jnp.float32),
            pltpu.VMEM((_RING * _L, 8, 128), jnp.float32),
            pltpu.VMEM((_L,), jnp.float32),
            pltpu.SemaphoreType.DMA,
            pltpu.SemaphoreType.DMA,
            pltpu.SemaphoreType.DMA,
        ],
    )
    def sc_call(i_hbm, j_hbm, x_hbm, w_hbm, wt_hbm, out_hbm,
                ii_v, jj_v, x_v, bufi, bufj, acc_v, sem0, sem1, sem2):
        wid = lax.axis_index("s") * _NC + lax.axis_index("c")
        base = wid * C
        pltpu.sync_copy(i_hbm.at[pl.ds(base, C)], ii_v)
        pltpu.sync_copy(j_hbm.at[pl.ds(base, C)], jj_v)
        pltpu.sync_copy(x_hbm.at[pl.ds(base, C)], x_v)

        iota = lax.iota(jnp.int32, _L)
        sems = (sem0, sem1, sem2)

        def fire(t, u):
            g = t // _RB
            r = t % _RB
            slot = u * _L
            sem = sems[u]
            iv = ii_v[pl.ds(g * _L, _L)]
            jv = jj_v[pl.ds(g * _L, _L)]
            vbi = (iv >> 7) * 128
            vbj = (jv >> 7) * 128
            for q in range(_L):
                oi = pl.multiple_of(vbi[q], 128)
                oj = pl.multiple_of(vbj[q], 128)
                pltpu.async_copy(w_hbm.at[r, :, pl.ds(oi, 128)],
                                 bufi.at[slot + q], sem)
                pltpu.async_copy(wt_hbm.at[r, :, pl.ds(oj, 128)],
                                 bufj.at[slot + q], sem)

        def drain(u):
            slot = u * _L
            sem = sems[u]
            dummy = w_hbm.at[:, :, pl.ds(0, 128)]
            for h in range(2):
                pltpu.make_async_copy(
                    dummy, bufi.at[pl.ds(slot + 8 * h, 8)], sem).wait()
                pltpu.make_async_copy(
                    dummy, bufj.at[pl.ds(slot + 8 * h, 8)], sem).wait()

        def do_phase(t, u, dots, acc):
            drain(u)
            g = t // _RB
            slot = u * _L + iota
            iv = ii_v[pl.ds(g * _L, _L)]
            jv = jj_v[pl.ds(g * _L, _L)]
            civ = iv & 127
            cjv = jv & 127
            for dr in range(8):
                drv = jnp.full((_L,), dr, jnp.int32)
                gi = plsc.load_gather(bufi, [slot, drv, civ])
                gj = plsc.load_gather(bufj, [slot, drv, cjv])
                dots = dots + gi * gj

            xg = x_v[pl.ds(g * _L, _L)]
            lnx = _ln(xg)
            lnw = jnp.minimum(lnx - _LN_XMAX, 0.0)
            weight = jnp.exp(jnp.float32(GLOVE_ALPHA) * lnw)
            diff = dots - lnx
            contrib = weight * diff * diff
            last = jnp.full((_L,), (t % _RB) == (_RB - 1))
            acc = acc + jnp.where(last, contrib, 0.0)
            dots = jnp.where(last, 0.0, dots)
            return dots, acc

        fire(0, 0)
        fire(1, 1)

        def body(i, carry):
            dots, acc = carry
            t0 = i * _RING
            for u in range(_RING):
                t = t0 + u

                @pl.when(t + 2 < T)
                def _(t=t, u=u):
                    fire(t + 2, (u + 2) % _RING)

                dots, acc = do_phase(t, u, dots, acc)
            return dots, acc

        nfull = (T - 1) // _RING
        dots, acc = lax.fori_loop(
            0, nfull, body,
            (jnp.zeros((_L,), jnp.float32), jnp.zeros((_L,), jnp.float32)))
        for t in range(nfull * _RING, T):
            _, acc = do_phase(t, t % _RING, dots, acc)
        acc_v[...] = acc
        pltpu.sync_copy(acc_v, out_hbm.at[wid])

    return sc_call


def kernel(i_idx, j_idx, x_ij, W, W_tilde, b, b_tilde):
    B = x_ij.shape[0]
    D = W.shape[1]
    sc_call = _make_sc_call(B, D)
    partials = sc_call(i_idx.astype(jnp.int32), j_idx.astype(jnp.int32),
                       x_ij, W.T.reshape(8, 8, W.shape[0]),
                       W_tilde.T.reshape(8, 8, W.shape[0]))
    return jnp.sum(partials) / jnp.float32(B)
